# baseline (speedup 1.0000x reference)
import functools

import jax
import jax.numpy as jnp
from jax import lax
from jax.experimental import pallas as pl
from jax.experimental.pallas import tpu as pltpu
from jax.experimental.pallas import tpu_sc as plsc

_B = 4
_S = 2048
_D = 1024
_NW = 32
_S_SC = 768
_S_TC = _S - _S_SC
_SROWS = _S_SC // _NW
_CROWS = _SROWS
_NJOBS = _B
_CHUNK = _CROWS * _D
_BS_TC = 256


def _sc_body(x_hbm, emb_hbm, out_hbm, emb_v, x_v0, x_v1, ld0, ld1, st0, st1):
    wid = lax.axis_index("s") * 2 + lax.axis_index("c")
    row0 = _S_TC + wid * _SROWS
    orow0 = wid * _SROWS
    x_slots = (x_v0, x_v1)
    ld_sems = (ld0, ld1)

    lds = {}
    sts = {}
    lds[0] = pltpu.async_copy(x_hbm.at[0, pl.ds(row0, _CROWS), :],
                              x_slots[0], ld_sems[0])
    pltpu.sync_copy(emb_hbm.at[pl.ds(row0, _CROWS), :], emb_v)
    for j in range(_NJOBS):
        slot = j % 2
        lds[j].wait()
        if j + 1 < _NJOBS:
            if j - 1 >= 0:
                sts[j - 1].wait()
            nslot = (j + 1) % 2
            lds[j + 1] = pltpu.async_copy(
                x_hbm.at[j + 1, pl.ds(row0, _CROWS), :],
                x_slots[nslot], ld_sems[nslot])

        xa = x_slots[slot]

        @plsc.parallel_loop(0, _CHUNK, 16, unroll=8)
        def _add(i):
            r = i >> 10
            c16 = pl.multiple_of(i & (_D - 1), 16)
            xa[r, pl.ds(c16, 16)] = (xa[r, pl.ds(c16, 16)]
                                     + emb_v[r, pl.ds(c16, 16)])

        sts[j] = pltpu.async_copy(x_slots[slot],
                                  out_hbm.at[j, pl.ds(orow0, _CROWS), :],
                                  (st0, st1)[slot])
    sts[_NJOBS - 2].wait()
    sts[_NJOBS - 1].wait()


@functools.partial(
    pl.kernel,
    mesh=plsc.VectorSubcoreMesh(core_axis_name="c", subcore_axis_name="s"),
    out_type=jax.ShapeDtypeStruct((_B, _S_SC, _D), jnp.float32),
    scratch_types=[
        pltpu.VMEM((_CROWS, _D), jnp.float32),
        pltpu.VMEM((_CROWS, _D), jnp.float32),
        pltpu.VMEM((_CROWS, _D), jnp.float32),
        pltpu.SemaphoreType.DMA,
        pltpu.SemaphoreType.DMA,
        pltpu.SemaphoreType.DMA,
        pltpu.SemaphoreType.DMA,
    ],
)
def _sc_add(x_hbm, emb_hbm, out_hbm, emb_v, x_v0, x_v1, ld0, ld1, st0, st1):
    _sc_body(x_hbm, emb_hbm, out_hbm, emb_v, x_v0, x_v1, ld0, ld1, st0, st1)


def _tc_body(x_ref, emb_ref, out_ref):
    out_ref[...] = x_ref[...] + emb_ref[...][None, :, :]


def _tc_add(x, emb):
    return pl.pallas_call(
        _tc_body,
        grid=(_S_TC // _BS_TC, _B),
        in_specs=[
            pl.BlockSpec((1, _BS_TC, _D), lambda s, b: (b, s, 0)),
            pl.BlockSpec((_BS_TC, _D), lambda s, b: (s, 0)),
        ],
        out_specs=pl.BlockSpec((1, _BS_TC, _D), lambda s, b: (b, s, 0)),
        out_shape=jax.ShapeDtypeStruct((_B, _S_TC, _D), x.dtype),
    )(x, emb)


def kernel(x, emb):
    out_sc = _sc_add(x, emb)
    out_tc = _tc_add(x, emb)
    return jnp.concatenate([out_tc, out_sc], axis=1)


# speedup vs baseline: 1.3495x; 1.3495x over previous
import functools

import jax
import jax.numpy as jnp
from jax import lax
from jax.experimental import pallas as pl
from jax.experimental.pallas import tpu as pltpu
from jax.experimental.pallas import tpu_sc as plsc

_B = 4
_S = 2048
_D = 1024
_NW = 32
_SROWS = _S // _NW
_CROWS = 8
_NCHUNK = _SROWS // _CROWS
_CHUNK = _CROWS * _D


def _sc_body(x_hbm, emb_hbm, out_hbm, emb_v, x_v, e_sems, l_sems, s_sems):
    wid = lax.axis_index("s") * 2 + lax.axis_index("c")
    row0 = wid * _SROWS

    def start_loads(c, slot):
        r = row0 + c * _CROWS
        elds = pltpu.async_copy(emb_hbm.at[pl.ds(r, _CROWS), :],
                                emb_v[slot], e_sems[slot])
        xlds = [pltpu.async_copy(x_hbm.at[b, pl.ds(r, _CROWS), :],
                                 x_v[b][slot], l_sems[b][slot])
                for b in range(_B)]
        return elds, xlds

    lds = {0: start_loads(0, 0)}
    sts = {}
    for c in range(_NCHUNK):
        slot = c % 2
        if c + 1 < _NCHUNK:
            if c - 1 >= 0:
                for h in sts[c - 1]:
                    h.wait()
            lds[c + 1] = start_loads(c + 1, (c + 1) % 2)
        elds, xlds = lds[c]
        elds.wait()
        for h in xlds:
            h.wait()

        ev = emb_v[slot]
        xa0, xa1, xa2, xa3 = (x_v[b][slot] for b in range(_B))

        @plsc.parallel_loop(0, _CHUNK, 16, unroll=8)
        def _add(i):
            r = i >> 10
            c16 = pl.multiple_of(i & (_D - 1), 16)
            e = ev[r, pl.ds(c16, 16)]
            xa0[r, pl.ds(c16, 16)] = xa0[r, pl.ds(c16, 16)] + e
            xa1[r, pl.ds(c16, 16)] = xa1[r, pl.ds(c16, 16)] + e
            xa2[r, pl.ds(c16, 16)] = xa2[r, pl.ds(c16, 16)] + e
            xa3[r, pl.ds(c16, 16)] = xa3[r, pl.ds(c16, 16)] + e

        r = row0 + c * _CROWS
        sts[c] = [pltpu.async_copy(x_v[b][slot],
                                   out_hbm.at[b, pl.ds(r, _CROWS), :],
                                   s_sems[b][slot])
                  for b in range(_B)]
    for c in (_NCHUNK - 2, _NCHUNK - 1):
        for h in sts[c]:
            h.wait()


_SCRATCH = (
    [pltpu.VMEM((_CROWS, _D), jnp.float32) for _ in range(2)]
    + [pltpu.VMEM((_CROWS, _D), jnp.float32) for _ in range(_B * 2)]
    + [pltpu.SemaphoreType.DMA for _ in range(2 + _B * 2 + _B * 2)]
)


@functools.partial(
    pl.kernel,
    mesh=plsc.VectorSubcoreMesh(core_axis_name="c", subcore_axis_name="s"),
    out_type=jax.ShapeDtypeStruct((_B, _S, _D), jnp.float32),
    scratch_types=_SCRATCH,
)
def _sc_add(x_hbm, emb_hbm, out_hbm, *scratch):
    emb_v = scratch[0:2]
    x_v = [scratch[2 + 2 * b:4 + 2 * b] for b in range(_B)]
    e_sems = scratch[10:12]
    l_sems = [scratch[12 + 2 * b:14 + 2 * b] for b in range(_B)]
    s_sems = [scratch[20 + 2 * b:22 + 2 * b] for b in range(_B)]
    _sc_body(x_hbm, emb_hbm, out_hbm, emb_v, x_v, e_sems, l_sems, s_sems)


def kernel(x, emb):
    B, S, D = x.shape
    return _sc_add(x, emb[:S])


# speedup vs baseline: 2.8722x; 2.1283x over previous
import jax
import jax.numpy as jnp
from jax.experimental import pallas as pl


def _add_body(x_ref, emb_ref, out_ref):
    out_ref[...] = x_ref[...] + emb_ref[...][None, :, :]


def kernel(x, emb):
    B, S, D = x.shape
    return pl.pallas_call(
        _add_body,
        grid=(B,),
        in_specs=[
            pl.BlockSpec((1, S, D), lambda b: (b, 0, 0)),
            pl.BlockSpec((S, D), lambda b: (0, 0)),
        ],
        out_specs=pl.BlockSpec((1, S, D), lambda b: (b, 0, 0)),
        out_shape=jax.ShapeDtypeStruct((B, S, D), x.dtype),
    )(x, emb[:S])
